# baseline (speedup 1.0000x reference)
import dataclasses
import functools

import jax
import jax.numpy as jnp
from jax import lax
from jax.experimental import pallas as pl
from jax.experimental.pallas import tpu as pltpu
from jax.experimental.pallas import tpu_sc as plsc

NC = 2
NS = 16
NW = NC * NS


def _sc_mesh():
    return plsc.VectorSubcoreMesh(core_axis_name="c", subcore_axis_name="s")


def _sc_no_layout_params():
    cp = pltpu.CompilerParams()
    if "needs_layout_passes" in pltpu.CompilerParams.__dataclass_fields__:
        cp = dataclasses.replace(cp, needs_layout_passes=False)
    return cp


def _sc_degrees(es, ed, zeros_n):
    e = es.shape[0]
    n = zeros_n.shape[0]
    per_tile = e // NW
    ch = 1000
    n_ch = per_tile // ch

    @functools.partial(
        pl.kernel,
        out_type=(jax.ShapeDtypeStruct((NC, 1, n), jnp.float32),
                  jax.ShapeDtypeStruct((NC, 1, n), jnp.float32)),
        mesh=_sc_mesh(),
        scratch_types=[
            pltpu.VMEM((ch,), jnp.int32),
            pltpu.VMEM((ch,), jnp.int32),
            pltpu.VMEM((ch,), jnp.int32),
            pltpu.VMEM((ch,), jnp.int32),
            pltpu.VMEM((ch,), jnp.float32),
            pltpu.VMEM_SHARED((n,), jnp.float32),
            pltpu.VMEM_SHARED((n,), jnp.float32),
            pltpu.SemaphoreType.DMA,
            pltpu.SemaphoreType.DMA,
        ],
    )
    def k(es_hbm, ed_hbm, z_hbm, od_hbm, id_hbm,
          es_a, ed_a, es_b, ed_b, ones_v, od_sh, id_sh, sem_a, sem_b):
        c = lax.axis_index("c")
        s = lax.axis_index("s")
        base = (c * NS + s) * per_tile

        def idx_start(j, ese, ede, sem):
            off = base + j * ch
            pltpu.async_copy(es_hbm.at[pl.ds(off, ch)], ese, sem)
            pltpu.async_copy(ed_hbm.at[pl.ds(off, ch)], ede, sem)

        def idx_wait(ese, ede, sem):
            pltpu.make_async_copy(es_hbm.at[pl.ds(base, ch)], ese, sem).wait()
            pltpu.make_async_copy(ed_hbm.at[pl.ds(base, ch)], ede, sem).wait()

        idx_start(0, es_a, ed_a, sem_a)
        if n_ch > 1:
            idx_start(1, es_b, ed_b, sem_b)

        @pl.loop(0, ch, step=16)
        def _(i):
            ones_v[pl.ds(i, 16)] = jnp.full((16,), 1.0, jnp.float32)

        @pl.when(s == 0)
        def _():
            pltpu.sync_copy(z_hbm, od_sh)
            pltpu.sync_copy(z_hbm, id_sh)

        plsc.subcore_barrier()

        @pl.loop(0, n_ch, step=2)
        def _(i):
            idx_wait(es_a, ed_a, sem_a)
            pltpu.sync_copy(ones_v, od_sh.at[es_a], add=True)
            pltpu.sync_copy(ones_v, id_sh.at[ed_a], add=True)

            @pl.when(i + 2 < n_ch)
            def _():
                idx_start(i + 2, es_a, ed_a, sem_a)

            @pl.when(i + 1 < n_ch)
            def _():
                idx_wait(es_b, ed_b, sem_b)
                pltpu.sync_copy(ones_v, od_sh.at[es_b], add=True)
                pltpu.sync_copy(ones_v, id_sh.at[ed_b], add=True)

                @pl.when(i + 3 < n_ch)
                def _():
                    idx_start(i + 3, es_b, ed_b, sem_b)

        plsc.subcore_barrier()

        @pl.when(s == 0)
        def _():
            pltpu.sync_copy(od_sh, od_hbm.at[c, 0])
            pltpu.sync_copy(id_sh, id_hbm.at[c, 0])

    return k(es, ed, zeros_n)


def _tc_norms_scale(od_p, id_p, x_p):
    n_pad, d = x_p.shape
    bn = 1024

    def body(od_ref, id_ref, x_ref, h0_ref, on_ref, in_ref):
        sl = pl.ds(pl.program_id(0) * bn, bn)
        od = od_ref[0, 0, :] + od_ref[1, 0, :]
        idg = id_ref[0, 0, :] + id_ref[1, 0, :]
        on = lax.rsqrt(jnp.maximum(od, 1.0))
        inn = lax.rsqrt(jnp.maximum(idg, 1.0))
        on_ref[sl] = on
        in_ref[sl] = inn
        h0_ref[...] = x_ref[...] * on[:, None]

    return pl.pallas_call(
        body,
        grid=(n_pad // bn,),
        in_specs=[
            pl.BlockSpec((NC, 1, bn), lambda i: (0, 0, i)),
            pl.BlockSpec((NC, 1, bn), lambda i: (0, 0, i)),
            pl.BlockSpec((bn, d), lambda i: (i, 0)),
        ],
        out_specs=[
            pl.BlockSpec((bn, d), lambda i: (i, 0)),
            pl.BlockSpec((n_pad,), lambda i: (0,)),
            pl.BlockSpec((n_pad,), lambda i: (0,)),
        ],
        out_shape=[
            jax.ShapeDtypeStruct((n_pad, d), jnp.float32),
            jax.ShapeDtypeStruct((n_pad,), jnp.float32),
            jax.ShapeDtypeStruct((n_pad,), jnp.float32),
        ],
    )(od_p, id_p, x_p)


def _sc_aggregate(h, es, ed, zeros_nd):
    n, d = h.shape
    e = es.shape[0]
    per_tile = e // NW
    ch = 184
    n_ch = per_tile // ch
    tail = per_tile - n_ch * ch
    rows_per_sub = n // NS

    @functools.partial(
        pl.kernel,
        out_type=jax.ShapeDtypeStruct((NC, n, d), jnp.float32),
        mesh=_sc_mesh(),
        scratch_types=[
            pltpu.VMEM((ch,), jnp.int32),
            pltpu.VMEM((ch,), jnp.int32),
            pltpu.VMEM((ch,), jnp.int32),
            pltpu.VMEM((ch,), jnp.int32),
            pltpu.VMEM((max(tail, 8),), jnp.int32),
            pltpu.VMEM((max(tail, 8),), jnp.int32),
            pltpu.VMEM((ch, d), jnp.float32),
            pltpu.VMEM((ch, d), jnp.float32),
            pltpu.VMEM_SHARED((n, d), jnp.float32),
            pltpu.SemaphoreType.DMA,
            pltpu.SemaphoreType.DMA,
            pltpu.SemaphoreType.DMA,
            pltpu.SemaphoreType.DMA,
        ],
    )
    def k(h_hbm, es_hbm, ed_hbm, z_hbm, out_hbm,
          es_a, ed_a, es_b, ed_b, es_t, ed_t, rows_a, rows_b, acc_sh,
          sem_ia, sem_ib, sem_ga, sem_gb):
        c = lax.axis_index("c")
        s = lax.axis_index("s")
        r0 = pl.multiple_of(s * rows_per_sub, 8)
        pltpu.sync_copy(z_hbm.at[pl.ds(r0, rows_per_sub)],
                        acc_sh.at[pl.ds(r0, rows_per_sub)])
        plsc.subcore_barrier()
        base = (c * NS + s) * per_tile

        def idx_start(j, ese, ede, sem):
            off = base + j * ch
            pltpu.async_copy(es_hbm.at[pl.ds(off, ch)], ese, sem)
            pltpu.async_copy(ed_hbm.at[pl.ds(off, ch)], ede, sem)

        def idx_wait(ese, ede, sem):
            pltpu.make_async_copy(es_hbm.at[pl.ds(base, ch)], ese, sem).wait()
            pltpu.make_async_copy(ed_hbm.at[pl.ds(base, ch)], ede, sem).wait()

        def gather_start(ese, rows, sem):
            pltpu.async_copy(h_hbm.at[ese], rows, sem)

        def gather_wait(ese, rows, sem):
            pltpu.make_async_copy(h_hbm.at[ese], rows, sem).wait()

        idx_start(0, es_a, ed_a, sem_ia)
        idx_wait(es_a, ed_a, sem_ia)
        gather_start(es_a, rows_a, sem_ga)
        idx_start(1, es_b, ed_b, sem_ib)

        @pl.loop(0, n_ch, step=2)
        def _(i):
            idx_wait(es_b, ed_b, sem_ib)
            gather_start(es_b, rows_b, sem_gb)
            gather_wait(es_a, rows_a, sem_ga)
            pltpu.sync_copy(rows_a, acc_sh.at[ed_a], add=True)

            @pl.when(i + 2 < n_ch)
            def _():
                idx_start(i + 2, es_a, ed_a, sem_ia)
                idx_wait(es_a, ed_a, sem_ia)
                gather_start(es_a, rows_a, sem_ga)

            gather_wait(es_b, rows_b, sem_gb)
            pltpu.sync_copy(rows_b, acc_sh.at[ed_b], add=True)

            @pl.when(i + 3 < n_ch)
            def _():
                idx_start(i + 3, es_b, ed_b, sem_ib)

        if tail:
            toff = base + n_ch * ch
            pltpu.sync_copy(es_hbm.at[pl.ds(toff, tail)], es_t)
            pltpu.sync_copy(ed_hbm.at[pl.ds(toff, tail)], ed_t)
            pltpu.async_copy(h_hbm.at[es_t], rows_a.at[pl.ds(0, tail)],
                             sem_ga).wait()
            pltpu.sync_copy(rows_a.at[pl.ds(0, tail)], acc_sh.at[ed_t],
                            add=True)

        plsc.subcore_barrier()
        pltpu.sync_copy(acc_sh.at[pl.ds(r0, rows_per_sub)],
                        out_hbm.at[c, pl.ds(r0, rows_per_sub)])

    return k(h, es, ed, zeros_nd)


def _tc_layer(agg_p, inorm, w, b2d, onorm=None):
    _, n_pad, d = agg_p.shape
    bn = 1024

    def body(p_ref, in_ref, w_ref, b_ref, *rest):
        if onorm is not None:
            on_ref, o_ref = rest
        else:
            (o_ref,) = rest
        sl = pl.ds(pl.program_id(0) * bn, bn)
        agg = (p_ref[0] + p_ref[1]) * in_ref[sl][:, None]
        y = jnp.dot(agg, w_ref[...], preferred_element_type=jnp.float32)
        y = y + b_ref[...]
        if onorm is not None:
            y = jnp.maximum(y, 0.0) * on_ref[sl][:, None]
        o_ref[...] = y

    in_specs = [
        pl.BlockSpec((NC, bn, d), lambda i: (0, i, 0)),
        pl.BlockSpec((n_pad,), lambda i: (0,)),
        pl.BlockSpec((d, d), lambda i: (0, 0)),
        pl.BlockSpec((1, d), lambda i: (0, 0)),
    ]
    args = [agg_p, inorm, w, b2d]
    if onorm is not None:
        in_specs.append(pl.BlockSpec((n_pad,), lambda i: (0,)))
        args.append(onorm)
    return pl.pallas_call(
        body,
        grid=(n_pad // bn,),
        in_specs=in_specs,
        out_specs=pl.BlockSpec((bn, d), lambda i: (i, 0)),
        out_shape=jax.ShapeDtypeStruct((n_pad, d), jnp.float32),
    )(*args)


def _sc_edge_scores(h, src, dst):
    n, d = h.shape
    q = src.shape[0]
    ch = 160
    n_ch = q // ch
    nd16 = d // 16

    @functools.partial(
        pl.kernel,
        out_type=jax.ShapeDtypeStruct((q,), jnp.float32),
        mesh=_sc_mesh(),
        compiler_params=_sc_no_layout_params(),
        scratch_types=[
            pltpu.VMEM((ch,), jnp.int32),
            pltpu.VMEM((ch,), jnp.int32),
            pltpu.VMEM((ch,), jnp.int32),
            pltpu.VMEM((ch,), jnp.int32),
            pltpu.VMEM((ch, d), jnp.float32),
            pltpu.VMEM((ch, d), jnp.float32),
            pltpu.VMEM((ch, d), jnp.float32),
            pltpu.VMEM((ch, d), jnp.float32),
            pltpu.VMEM((ch, 16), jnp.float32),
            pltpu.VMEM((ch,), jnp.float32),
            pltpu.SemaphoreType.DMA,
            pltpu.SemaphoreType.DMA,
            pltpu.SemaphoreType.DMA,
            pltpu.SemaphoreType.DMA,
        ],
    )
    def k(h_hbm, src_hbm, dst_hbm, out_hbm,
          si_a, di_a, si_b, di_b, srows_a, drows_a, srows_b, drows_b,
          cum_v, out_v, sem_ia, sem_ib, sem_ga, sem_gb):
        c = lax.axis_index("c")
        s = lax.axis_index("s")
        wid = c * NS + s
        n_my = (n_ch - wid + NW - 1) // NW

        def chunk_of(k_):
            return wid + k_ * NW

        def idx_start(j, si, di, sem):
            off = j * ch
            pltpu.async_copy(src_hbm.at[pl.ds(off, ch)], si, sem)
            pltpu.async_copy(dst_hbm.at[pl.ds(off, ch)], di, sem)

        def idx_wait(si, di, sem):
            pltpu.make_async_copy(src_hbm.at[pl.ds(0, ch)], si, sem).wait()
            pltpu.make_async_copy(dst_hbm.at[pl.ds(0, ch)], di, sem).wait()

        def g_start(si, di, srows, drows, sem):
            pltpu.async_copy(h_hbm.at[si], srows, sem)
            pltpu.async_copy(h_hbm.at[di], drows, sem)

        def g_wait(si, di, srows, drows, sem):
            pltpu.make_async_copy(h_hbm.at[si], srows, sem).wait()
            pltpu.make_async_copy(h_hbm.at[di], drows, sem).wait()

        cols15 = jnp.full((16,), 15, jnp.int32)

        def compute_and_store(j, srows, drows):
            @plsc.parallel_loop(0, ch, unroll=4)
            def _(qq):
                p = srows[qq, pl.ds(0, 16)] * drows[qq, pl.ds(0, 16)]
                for t in range(1, nd16):
                    p = p + (srows[qq, pl.ds(t * 16, 16)]
                             * drows[qq, pl.ds(t * 16, 16)])
                cum_v[qq, pl.ds(0, 16)] = plsc.cumsum(p)

            @plsc.parallel_loop(0, ch, step=16, unroll=2)
            def _(q0):
                qv = q0 + lax.iota(jnp.int32, 16)
                v = plsc.load_gather(cum_v, [qv, cols15])
                out_v[pl.ds(q0, 16)] = 1.0 / (1.0 + jnp.exp(-v))

            pltpu.sync_copy(out_v, out_hbm.at[pl.ds(j * ch, ch)])

        @pl.when(n_my > 0)
        def _():
            idx_start(chunk_of(0), si_a, di_a, sem_ia)
            idx_wait(si_a, di_a, sem_ia)
            g_start(si_a, di_a, srows_a, drows_a, sem_ga)

            @pl.when(n_my > 1)
            def _():
                idx_start(chunk_of(1), si_b, di_b, sem_ib)

            @pl.loop(0, n_my, step=2)
            def _(k_):
                @pl.when(k_ + 1 < n_my)
                def _():
                    idx_wait(si_b, di_b, sem_ib)
                    g_start(si_b, di_b, srows_b, drows_b, sem_gb)

                g_wait(si_a, di_a, srows_a, drows_a, sem_ga)
                compute_and_store(chunk_of(k_), srows_a, drows_a)

                @pl.when(k_ + 2 < n_my)
                def _():
                    idx_start(chunk_of(k_ + 2), si_a, di_a, sem_ia)
                    idx_wait(si_a, di_a, sem_ia)
                    g_start(si_a, di_a, srows_a, drows_a, sem_ga)

                @pl.when(k_ + 1 < n_my)
                def _():
                    g_wait(si_b, di_b, srows_b, drows_b, sem_gb)
                    compute_and_store(chunk_of(k_ + 1), srows_b, drows_b)

                    @pl.when(k_ + 3 < n_my)
                    def _():
                        idx_start(chunk_of(k_ + 3), si_b, di_b, sem_ib)

    return k(h, src, dst)


def kernel(x, edge_index, src, dst, W1, b1, W2, b2):
    n, d = x.shape
    n_pad = (n + 2047) // 2048 * 2048
    es = edge_index[0]
    ed = edge_index[1]
    x_p = jnp.pad(x, ((0, n_pad - n), (0, 0)))
    zeros_n = jnp.zeros((n_pad,), jnp.float32)
    zeros_nd = jnp.zeros((n_pad, d), jnp.float32)

    od_p, id_p = _sc_degrees(es, ed, zeros_n)
    h0, onorm, inorm = _tc_norms_scale(od_p, id_p, x_p)
    agg1 = _sc_aggregate(h0, es, ed, zeros_nd)
    h1 = _tc_layer(agg1, inorm, W1, b1.reshape(1, d), onorm)
    agg2 = _sc_aggregate(h1, es, ed, zeros_nd)
    h2 = _tc_layer(agg2, inorm, W2, b2.reshape(1, d))

    return _sc_edge_scores(h2, src, dst)


# speedup vs baseline: 1.0470x; 1.0470x over previous
import dataclasses
import functools

import jax
import jax.numpy as jnp
from jax import lax
from jax.experimental import pallas as pl
from jax.experimental.pallas import tpu as pltpu
from jax.experimental.pallas import tpu_sc as plsc

NC = 2
NS = 16
NW = NC * NS


def _sc_mesh():
    return plsc.VectorSubcoreMesh(core_axis_name="c", subcore_axis_name="s")


def _sc_no_layout_params():
    cp = pltpu.CompilerParams()
    if "needs_layout_passes" in pltpu.CompilerParams.__dataclass_fields__:
        cp = dataclasses.replace(cp, needs_layout_passes=False)
    return cp


def _sc_degrees(es, ed, zeros_n):
    e = es.shape[0]
    n = zeros_n.shape[0]
    per_tile = e // NW
    ch = 1000
    n_ch = per_tile // ch

    @functools.partial(
        pl.kernel,
        out_type=(jax.ShapeDtypeStruct((NC, 1, n), jnp.float32),
                  jax.ShapeDtypeStruct((NC, 1, n), jnp.float32)),
        mesh=_sc_mesh(),
        scratch_types=[
            pltpu.VMEM((ch,), jnp.int32),
            pltpu.VMEM((ch,), jnp.int32),
            pltpu.VMEM((ch,), jnp.int32),
            pltpu.VMEM((ch,), jnp.int32),
            pltpu.VMEM((ch,), jnp.float32),
            pltpu.VMEM_SHARED((n,), jnp.float32),
            pltpu.VMEM_SHARED((n,), jnp.float32),
            pltpu.SemaphoreType.DMA,
            pltpu.SemaphoreType.DMA,
        ],
    )
    def k(es_hbm, ed_hbm, z_hbm, od_hbm, id_hbm,
          es_a, ed_a, es_b, ed_b, ones_v, od_sh, id_sh, sem_a, sem_b):
        c = lax.axis_index("c")
        s = lax.axis_index("s")
        base = (c * NS + s) * per_tile

        def idx_start(j, ese, ede, sem):
            off = base + j * ch
            pltpu.async_copy(es_hbm.at[pl.ds(off, ch)], ese, sem)
            pltpu.async_copy(ed_hbm.at[pl.ds(off, ch)], ede, sem)

        def idx_wait(ese, ede, sem):
            pltpu.make_async_copy(es_hbm.at[pl.ds(base, ch)], ese, sem).wait()
            pltpu.make_async_copy(ed_hbm.at[pl.ds(base, ch)], ede, sem).wait()

        idx_start(0, es_a, ed_a, sem_a)
        if n_ch > 1:
            idx_start(1, es_b, ed_b, sem_b)

        @pl.loop(0, ch, step=16)
        def _(i):
            ones_v[pl.ds(i, 16)] = jnp.full((16,), 1.0, jnp.float32)

        @pl.when(s == 0)
        def _():
            pltpu.sync_copy(z_hbm, od_sh)
            pltpu.sync_copy(z_hbm, id_sh)

        plsc.subcore_barrier()

        @pl.loop(0, n_ch, step=2)
        def _(i):
            idx_wait(es_a, ed_a, sem_a)
            pltpu.sync_copy(ones_v, od_sh.at[es_a], add=True)
            pltpu.sync_copy(ones_v, id_sh.at[ed_a], add=True)

            @pl.when(i + 2 < n_ch)
            def _():
                idx_start(i + 2, es_a, ed_a, sem_a)

            @pl.when(i + 1 < n_ch)
            def _():
                idx_wait(es_b, ed_b, sem_b)
                pltpu.sync_copy(ones_v, od_sh.at[es_b], add=True)
                pltpu.sync_copy(ones_v, id_sh.at[ed_b], add=True)

                @pl.when(i + 3 < n_ch)
                def _():
                    idx_start(i + 3, es_b, ed_b, sem_b)

        plsc.subcore_barrier()

        @pl.when(s == 0)
        def _():
            pltpu.sync_copy(od_sh, od_hbm.at[c, 0])
            pltpu.sync_copy(id_sh, id_hbm.at[c, 0])

    return k(es, ed, zeros_n)


def _tc_norms_scale(od_p, id_p, x_p):
    n_pad, d = x_p.shape
    bn = 2048

    def body(od_ref, id_ref, x_ref, h0_ref, on_ref, in_ref):
        sl = pl.ds(pl.program_id(0) * bn, bn)
        od = od_ref[0, 0, :] + od_ref[1, 0, :]
        idg = id_ref[0, 0, :] + id_ref[1, 0, :]
        on = lax.rsqrt(jnp.maximum(od, 1.0))
        inn = lax.rsqrt(jnp.maximum(idg, 1.0))
        on_ref[sl] = on
        in_ref[sl] = inn
        h0_ref[...] = x_ref[...] * on[:, None]

    return pl.pallas_call(
        body,
        grid=(n_pad // bn,),
        in_specs=[
            pl.BlockSpec((NC, 1, bn), lambda i: (0, 0, i)),
            pl.BlockSpec((NC, 1, bn), lambda i: (0, 0, i)),
            pl.BlockSpec((bn, d), lambda i: (i, 0)),
        ],
        out_specs=[
            pl.BlockSpec((bn, d), lambda i: (i, 0)),
            pl.BlockSpec((n_pad,), lambda i: (0,)),
            pl.BlockSpec((n_pad,), lambda i: (0,)),
        ],
        out_shape=[
            jax.ShapeDtypeStruct((n_pad, d), jnp.float32),
            jax.ShapeDtypeStruct((n_pad,), jnp.float32),
            jax.ShapeDtypeStruct((n_pad,), jnp.float32),
        ],
    )(od_p, id_p, x_p)


def _sc_aggregate(h, es, ed):
    n, d = h.shape
    e = es.shape[0]
    per_tile = e // NW
    ch = 184
    n_ch = per_tile // ch
    tail = per_tile - n_ch * ch
    rows_per_sub = n // NS

    @functools.partial(
        pl.kernel,
        out_type=jax.ShapeDtypeStruct((NC, n, d), jnp.float32),
        mesh=_sc_mesh(),
        scratch_types=[
            pltpu.VMEM((ch,), jnp.int32),
            pltpu.VMEM((ch,), jnp.int32),
            pltpu.VMEM((ch,), jnp.int32),
            pltpu.VMEM((ch,), jnp.int32),
            pltpu.VMEM((max(tail, 8),), jnp.int32),
            pltpu.VMEM((max(tail, 8),), jnp.int32),
            pltpu.VMEM((ch, d), jnp.float32),
            pltpu.VMEM((ch, d), jnp.float32),
            pltpu.VMEM_SHARED((n, d), jnp.float32),
            pltpu.SemaphoreType.DMA,
            pltpu.SemaphoreType.DMA,
            pltpu.SemaphoreType.DMA,
            pltpu.SemaphoreType.DMA,
        ],
    )
    def k(h_hbm, es_hbm, ed_hbm, out_hbm,
          es_a, ed_a, es_b, ed_b, es_t, ed_t, rows_a, rows_b, acc_sh,
          sem_ia, sem_ib, sem_ga, sem_gb):
        c = lax.axis_index("c")
        s = lax.axis_index("s")
        r0 = pl.multiple_of(s * rows_per_sub, 8)

        zv = jnp.zeros((16,), jnp.float32)

        @pl.loop(0, ch)
        def _(r):
            for dd in range(d // 16):
                rows_a[r, pl.ds(dd * 16, 16)] = zv

        done = 0
        while done < rows_per_sub:
            step = min(ch, rows_per_sub - done)
            pltpu.sync_copy(
                rows_a.at[pl.ds(0, step)],
                acc_sh.at[pl.ds(pl.multiple_of(r0 + done, 8), step)])
            done += step
        plsc.subcore_barrier()
        base = (c * NS + s) * per_tile

        def idx_start(j, ese, ede, sem):
            off = base + j * ch
            pltpu.async_copy(es_hbm.at[pl.ds(off, ch)], ese, sem)
            pltpu.async_copy(ed_hbm.at[pl.ds(off, ch)], ede, sem)

        def idx_wait(ese, ede, sem):
            pltpu.make_async_copy(es_hbm.at[pl.ds(base, ch)], ese, sem).wait()
            pltpu.make_async_copy(ed_hbm.at[pl.ds(base, ch)], ede, sem).wait()

        def gather_start(ese, rows, sem):
            pltpu.async_copy(h_hbm.at[ese], rows, sem)

        def gather_wait(ese, rows, sem):
            pltpu.make_async_copy(h_hbm.at[ese], rows, sem).wait()

        idx_start(0, es_a, ed_a, sem_ia)
        idx_wait(es_a, ed_a, sem_ia)
        gather_start(es_a, rows_a, sem_ga)
        idx_start(1, es_b, ed_b, sem_ib)

        @pl.loop(0, n_ch, step=2)
        def _(i):
            idx_wait(es_b, ed_b, sem_ib)
            gather_start(es_b, rows_b, sem_gb)
            gather_wait(es_a, rows_a, sem_ga)
            pltpu.sync_copy(rows_a, acc_sh.at[ed_a], add=True)

            @pl.when(i + 2 < n_ch)
            def _():
                idx_start(i + 2, es_a, ed_a, sem_ia)
                idx_wait(es_a, ed_a, sem_ia)
                gather_start(es_a, rows_a, sem_ga)

            gather_wait(es_b, rows_b, sem_gb)
            pltpu.sync_copy(rows_b, acc_sh.at[ed_b], add=True)

            @pl.when(i + 3 < n_ch)
            def _():
                idx_start(i + 3, es_b, ed_b, sem_ib)

        if tail:
            toff = base + n_ch * ch
            pltpu.sync_copy(es_hbm.at[pl.ds(toff, tail)], es_t)
            pltpu.sync_copy(ed_hbm.at[pl.ds(toff, tail)], ed_t)
            pltpu.async_copy(h_hbm.at[es_t], rows_a.at[pl.ds(0, tail)],
                             sem_ga).wait()
            pltpu.sync_copy(rows_a.at[pl.ds(0, tail)], acc_sh.at[ed_t],
                            add=True)

        plsc.subcore_barrier()
        pltpu.sync_copy(acc_sh.at[pl.ds(r0, rows_per_sub)],
                        out_hbm.at[c, pl.ds(r0, rows_per_sub)])

    return k(h, es, ed)


def _tc_layer(agg_p, inorm, w, b2d, onorm=None):
    _, n_pad, d = agg_p.shape
    bn = 2048

    def body(p_ref, in_ref, w_ref, b_ref, *rest):
        if onorm is not None:
            on_ref, o_ref = rest
        else:
            (o_ref,) = rest
        sl = pl.ds(pl.program_id(0) * bn, bn)
        agg = (p_ref[0] + p_ref[1]) * in_ref[sl][:, None]
        y = jnp.dot(agg, w_ref[...], preferred_element_type=jnp.float32)
        y = y + b_ref[...]
        if onorm is not None:
            y = jnp.maximum(y, 0.0) * on_ref[sl][:, None]
        o_ref[...] = y

    in_specs = [
        pl.BlockSpec((NC, bn, d), lambda i: (0, i, 0)),
        pl.BlockSpec((n_pad,), lambda i: (0,)),
        pl.BlockSpec((d, d), lambda i: (0, 0)),
        pl.BlockSpec((1, d), lambda i: (0, 0)),
    ]
    args = [agg_p, inorm, w, b2d]
    if onorm is not None:
        in_specs.append(pl.BlockSpec((n_pad,), lambda i: (0,)))
        args.append(onorm)
    return pl.pallas_call(
        body,
        grid=(n_pad // bn,),
        in_specs=in_specs,
        out_specs=pl.BlockSpec((bn, d), lambda i: (i, 0)),
        out_shape=jax.ShapeDtypeStruct((n_pad, d), jnp.float32),
    )(*args)


def _sc_edge_scores(h, src, dst):
    n, d = h.shape
    q = src.shape[0]
    ch = 160
    n_ch = q // ch
    nd16 = d // 16

    @functools.partial(
        pl.kernel,
        out_type=jax.ShapeDtypeStruct((q,), jnp.float32),
        mesh=_sc_mesh(),
        compiler_params=_sc_no_layout_params(),
        scratch_types=[
            pltpu.VMEM((ch,), jnp.int32),
            pltpu.VMEM((ch,), jnp.int32),
            pltpu.VMEM((ch,), jnp.int32),
            pltpu.VMEM((ch,), jnp.int32),
            pltpu.VMEM((ch, d), jnp.float32),
            pltpu.VMEM((ch, d), jnp.float32),
            pltpu.VMEM((ch, d), jnp.float32),
            pltpu.VMEM((ch, d), jnp.float32),
            pltpu.VMEM((ch, 16), jnp.float32),
            pltpu.VMEM((ch,), jnp.float32),
            pltpu.SemaphoreType.DMA,
            pltpu.SemaphoreType.DMA,
            pltpu.SemaphoreType.DMA,
            pltpu.SemaphoreType.DMA,
        ],
    )
    def k(h_hbm, src_hbm, dst_hbm, out_hbm,
          si_a, di_a, si_b, di_b, srows_a, drows_a, srows_b, drows_b,
          cum_v, out_v, sem_ia, sem_ib, sem_ga, sem_gb):
        c = lax.axis_index("c")
        s = lax.axis_index("s")
        wid = c * NS + s
        n_my = (n_ch - wid + NW - 1) // NW

        def chunk_of(k_):
            return wid + k_ * NW

        def idx_start(j, si, di, sem):
            off = j * ch
            pltpu.async_copy(src_hbm.at[pl.ds(off, ch)], si, sem)
            pltpu.async_copy(dst_hbm.at[pl.ds(off, ch)], di, sem)

        def idx_wait(si, di, sem):
            pltpu.make_async_copy(src_hbm.at[pl.ds(0, ch)], si, sem).wait()
            pltpu.make_async_copy(dst_hbm.at[pl.ds(0, ch)], di, sem).wait()

        def g_start(si, di, srows, drows, sem):
            pltpu.async_copy(h_hbm.at[si], srows, sem)
            pltpu.async_copy(h_hbm.at[di], drows, sem)

        def g_wait(si, di, srows, drows, sem):
            pltpu.make_async_copy(h_hbm.at[si], srows, sem).wait()
            pltpu.make_async_copy(h_hbm.at[di], drows, sem).wait()

        cols15 = jnp.full((16,), 15, jnp.int32)

        def compute_and_store(j, srows, drows):
            @plsc.parallel_loop(0, ch, unroll=4)
            def _(qq):
                p = srows[qq, pl.ds(0, 16)] * drows[qq, pl.ds(0, 16)]
                for t in range(1, nd16):
                    p = p + (srows[qq, pl.ds(t * 16, 16)]
                             * drows[qq, pl.ds(t * 16, 16)])
                cum_v[qq, pl.ds(0, 16)] = plsc.cumsum(p)

            @plsc.parallel_loop(0, ch, step=16, unroll=2)
            def _(q0):
                qv = q0 + lax.iota(jnp.int32, 16)
                v = plsc.load_gather(cum_v, [qv, cols15])
                out_v[pl.ds(q0, 16)] = 1.0 / (1.0 + jnp.exp(-v))

            pltpu.sync_copy(out_v, out_hbm.at[pl.ds(j * ch, ch)])

        @pl.when(n_my > 0)
        def _():
            idx_start(chunk_of(0), si_a, di_a, sem_ia)
            idx_wait(si_a, di_a, sem_ia)
            g_start(si_a, di_a, srows_a, drows_a, sem_ga)

            @pl.when(n_my > 1)
            def _():
                idx_start(chunk_of(1), si_b, di_b, sem_ib)

            @pl.loop(0, n_my, step=2)
            def _(k_):
                @pl.when(k_ + 1 < n_my)
                def _():
                    idx_wait(si_b, di_b, sem_ib)
                    g_start(si_b, di_b, srows_b, drows_b, sem_gb)

                g_wait(si_a, di_a, srows_a, drows_a, sem_ga)
                compute_and_store(chunk_of(k_), srows_a, drows_a)

                @pl.when(k_ + 2 < n_my)
                def _():
                    idx_start(chunk_of(k_ + 2), si_a, di_a, sem_ia)
                    idx_wait(si_a, di_a, sem_ia)
                    g_start(si_a, di_a, srows_a, drows_a, sem_ga)

                @pl.when(k_ + 1 < n_my)
                def _():
                    g_wait(si_b, di_b, srows_b, drows_b, sem_gb)
                    compute_and_store(chunk_of(k_ + 1), srows_b, drows_b)

                    @pl.when(k_ + 3 < n_my)
                    def _():
                        idx_start(chunk_of(k_ + 3), si_b, di_b, sem_ib)

    return k(h, src, dst)


def kernel(x, edge_index, src, dst, W1, b1, W2, b2):
    n, d = x.shape
    n_pad = (n + 2047) // 2048 * 2048
    es = edge_index[0]
    ed = edge_index[1]
    x_p = jnp.pad(x, ((0, n_pad - n), (0, 0)))
    zeros_n = jnp.zeros((n_pad,), jnp.float32)

    od_p, id_p = _sc_degrees(es, ed, zeros_n)
    h0, onorm, inorm = _tc_norms_scale(od_p, id_p, x_p)
    agg1 = _sc_aggregate(h0, es, ed)
    h1 = _tc_layer(agg1, inorm, W1, b1.reshape(1, d), onorm)
    agg2 = _sc_aggregate(h1, es, ed)
    h2 = _tc_layer(agg2, inorm, W2, b2.reshape(1, d))

    return _sc_edge_scores(h2, src, dst)
